# baseline (speedup 1.0000x reference)
import functools

import jax
import jax.numpy as jnp
from jax import lax
from jax.experimental import pallas as pl
from jax.experimental.pallas import tpu as pltpu
from jax.experimental.pallas import tpu_sc as plsc

N_NODES = 10000
N_EDGES = 160000
D_FEAT = 128
D_EDGE = 16
F_OUT = 32

NC = 2
NS = 16
NW = NC * NS
CHUNK = 128
E_PAD = 163840
N_CHUNKS = E_PAD // (NW * CHUNK)
N_PAD = 10240
TRASH = N_NODES
ROWS_PER_TILE = N_PAD // NS

EB = 8192
NB4 = 512
NBUF = 4


CB_FAST = 40
CB_SLOW = 40


def _make_sc_gather(fin):
  mesh = plsc.VectorSubcoreMesh(core_axis_name="c", subcore_axis_name="s")

  @functools.partial(
      pl.kernel, mesh=mesh,
      out_type=jax.ShapeDtypeStruct((E_PAD, fin), jnp.float32),
      compiler_params=pltpu.CompilerParams(use_tc_tiling_on_sc=False),
      scratch_types=[
          pltpu.VMEM((CB_FAST, CHUNK), jnp.int32),
      ] + [pltpu.VMEM((CHUNK, fin), jnp.float32) for _ in range(NBUF)]
        + [pltpu.SemaphoreType.DMA for _ in range(NBUF)],
  )
  def gather_k(table_hbm, idx_hbm, out_hbm, idx_v, *rest):
    bufs, sems = rest[:NBUF], rest[NBUF:]
    cid = lax.axis_index("c")
    sid = lax.axis_index("s")

    def run(base, count):
      pltpu.sync_copy(idx_hbm.at[pl.ds(base, count)],
                      idx_v.at[pl.ds(0, count)])
      for b in range(NBUF):
        pltpu.async_copy(table_hbm.at[idx_v.at[b]], bufs[b], sems[b])

      def body(g, carry):
        for b in range(NBUF):
          j = g * NBUF + b
          pltpu.make_async_copy(table_hbm.at[idx_v.at[j]], bufs[b],
                                sems[b]).wait()
          pltpu.sync_copy(bufs[b],
                          out_hbm.at[pl.ds((base + j) * CHUNK, CHUNK)])

          @pl.when(j + NBUF < count)
          def _():
            pltpu.async_copy(table_hbm.at[idx_v.at[j + NBUF]], bufs[b],
                             sems[b])

        return carry

      lax.fori_loop(0, count // NBUF, body, 0)

    @pl.when(cid == 1)
    def _():
      run(sid * CB_FAST, CB_FAST)

    @pl.when(cid == 0)
    def _():
      run(NS * CB_FAST + sid * CB_SLOW, CB_SLOW)

  return gather_k


def _make_sc_scatter():
  mesh = plsc.VectorSubcoreMesh(core_axis_name="c", subcore_axis_name="s")

  @functools.partial(
      pl.kernel, mesh=mesh,
      out_type=jax.ShapeDtypeStruct((NC, N_PAD, F_OUT), jnp.float32),
      compiler_params=pltpu.CompilerParams(use_tc_tiling_on_sc=False),
      scratch_types=[
          pltpu.VMEM((N_CHUNKS, CHUNK), jnp.int32),
          pltpu.VMEM_SHARED((N_PAD, F_OUT), jnp.float32),
      ] + [pltpu.VMEM((CHUNK, F_OUT), jnp.float32) for _ in range(NBUF)]
        + [pltpu.SemaphoreType.DMA for _ in range(NBUF)],
  )
  def scatter_k(msg_hbm, dst_hbm, zeros_hbm, out_hbm, idx_v, acc_sh, *rest):
    bufs, sems = rest[:NBUF], rest[NBUF:]
    cid = lax.axis_index("c")
    sid = lax.axis_index("s")
    wid = sid * NC + cid
    r0 = sid * ROWS_PER_TILE
    base = wid * N_CHUNKS

    pltpu.sync_copy(zeros_hbm.at[pl.ds(r0, ROWS_PER_TILE)],
                    acc_sh.at[pl.ds(r0, ROWS_PER_TILE)])
    pltpu.sync_copy(dst_hbm.at[wid], idx_v)
    for b in range(NBUF):
      pltpu.async_copy(msg_hbm.at[pl.ds((base + b) * CHUNK, CHUNK)],
                       bufs[b], sems[b])
    plsc.subcore_barrier()

    def body(g, carry):
      for b in range(NBUF):
        j = g * NBUF + b
        pltpu.make_async_copy(
            msg_hbm.at[pl.ds((base + j) * CHUNK, CHUNK)], bufs[b],
            sems[b]).wait()
        pltpu.sync_copy(bufs[b], acc_sh.at[idx_v.at[j]], add=True)

        @pl.when(j + NBUF < N_CHUNKS)
        def _():
          pltpu.async_copy(
              msg_hbm.at[pl.ds((base + j + NBUF) * CHUNK, CHUNK)],
              bufs[b], sems[b])

      return carry

    lax.fori_loop(0, N_CHUNKS // NBUF, body, 0)

    plsc.subcore_barrier()
    pltpu.sync_copy(acc_sh.at[pl.ds(r0, ROWS_PER_TILE)],
                    out_hbm.at[cid, pl.ds(r0, ROWS_PER_TILE)])

  return scatter_k


EAB = 8192


def _ea_pack_body(ea_ref, out_ref):
  e3 = ea_ref[...].reshape(EAB // 4, 4, D_EDGE)
  out_ref[...] = jnp.concatenate(
      [e3[:, j, :] for j in range(4)], axis=1).astype(jnp.bfloat16)


def _tc_ea_pack(ea):
  grid = (N_EDGES + EAB - 1) // EAB
  return pl.pallas_call(
      _ea_pack_body,
      grid=(grid,),
      in_specs=[pl.BlockSpec((EAB, D_EDGE), lambda i: (i, 0))],
      out_specs=pl.BlockSpec((EAB // 4, 4 * D_EDGE), lambda i: (i, 0)),
      out_shape=jax.ShapeDtypeStruct((E_PAD // 4, 4 * D_EDGE), jnp.bfloat16),
  )(ea)


def _msg_body(fin, xs_ref, ea_ref, u_ref, bk_ref, r_ref, s_ref, out_ref):
  xs4 = xs_ref[...]
  if fin == 128:
    xs4 = xs4.reshape(EB // 4, 4 * fin)
  xb = xs4.astype(jnp.bfloat16)
  eab = ea_ref[...]
  acc = jnp.dot(xs4, bk_ref[...], preferred_element_type=jnp.float32)
  ys = [jnp.dot(xb[:, g * fin:(g + 1) * fin], u_ref[...],
                preferred_element_type=jnp.float32) for g in range(4)]
  as_ = [jnp.dot(eab, r_ref[:, g * 512:(g + 1) * 512],
                 preferred_element_type=jnp.float32) for g in range(4)]
  for g in range(4):
    acc = acc + jnp.dot(as_[g] * ys[g], s_ref[pl.ds(g * 512, 512), :],
                        preferred_element_type=jnp.float32)
  out_ref[...] = acc


def _tc_messages(xs, ea4, u, bk4, r4, s4):
  fin = u.shape[0]
  xs_block = (EB, 128) if fin == 128 else (EB // 4, 128)
  cn = lambda i: (0, 0)
  return pl.pallas_call(
      functools.partial(_msg_body, fin),
      grid=(E_PAD // EB,),
      in_specs=[
          pl.BlockSpec(xs_block, lambda i: (i, 0)),
          pl.BlockSpec((EB // 4, 64), lambda i: (i, 0)),
          pl.BlockSpec(u.shape, cn),
          pl.BlockSpec(bk4.shape, cn),
          pl.BlockSpec(r4.shape, cn),
          pl.BlockSpec(s4.shape, cn),
      ],
      out_specs=pl.BlockSpec((EB // 4, 128), lambda i: (i, 0)),
      out_shape=jax.ShapeDtypeStruct((E_PAD // 4, 128), jnp.float32),
      compiler_params=pltpu.CompilerParams(vmem_limit_bytes=100 * 1024 * 1024),
  )(xs, ea4, u, bk4, r4, s4)


def _root_body(p_ref, x_ref, w_ref, b_ref, out_ref):
  agg = p_ref[0] + p_ref[1]
  z = agg + jnp.dot(x_ref[...], w_ref[...],
                    preferred_element_type=jnp.float32) + b_ref[...]
  out_ref[...] = jnp.maximum(z, 0.0)


def _tc_root(parts4, x4, w4root, b4):
  fin4 = x4.shape[1]
  return pl.pallas_call(
      _root_body,
      grid=(N_PAD // (4 * NB4),),
      in_specs=[
          pl.BlockSpec((NC, NB4, 128), lambda i: (0, i, 0)),
          pl.BlockSpec((NB4, fin4), lambda i: (i, 0)),
          pl.BlockSpec(w4root.shape, lambda i: (0, 0)),
          pl.BlockSpec((1, 128), lambda i: (0, 0)),
      ],
      out_specs=pl.BlockSpec((NB4, 128), lambda i: (i, 0)),
      out_shape=jax.ShapeDtypeStruct((N_PAD // 4, 128), jnp.float32),
  )(parts4, x4, w4root, b4)


def _final_body(p_ref, x_ref, w_ref, b_ref, w1_ref, c1_ref, w2_ref, c2_ref,
                w3_ref, c3_ref, w4_ref, c4_ref, out_ref, acc_ref):
  i = pl.program_id(0)
  agg = p_ref[0] + p_ref[1]
  h = jnp.maximum(
      agg + jnp.dot(x_ref[...], w_ref[...],
                    preferred_element_type=jnp.float32) + b_ref[...], 0.0)
  row = i * NB4 + lax.broadcasted_iota(jnp.int32, (NB4, 1), 0)
  h = jnp.where(row < N_NODES // 4, h, 0.0)
  psum = jnp.sum(h, axis=0, keepdims=True)

  @pl.when(i == 0)
  def _():
    acc_ref[...] = jnp.zeros_like(acc_ref)

  acc_ref[...] += psum

  @pl.when(i == pl.num_programs(0) - 1)
  def _():
    a = acc_ref[...]
    pooled = (a[:, 0:32] + a[:, 32:64] + a[:, 64:96] + a[:, 96:128])
    z = jnp.maximum(jnp.dot(pooled, w1_ref[...],
                            preferred_element_type=jnp.float32) + c1_ref[...], 0.0)
    z = jnp.maximum(jnp.dot(z, w2_ref[...],
                            preferred_element_type=jnp.float32) + c2_ref[...], 0.0)
    z = jnp.maximum(jnp.dot(z, w3_ref[...],
                            preferred_element_type=jnp.float32) + c3_ref[...], 0.0)
    out_ref[...] = jnp.sum(z * w4_ref[...]).reshape(1, 1) + c4_ref[...]


def _tc_final(parts4, x4, w4root, b4, w1, c1, w2, c2, w3, c3, w4r, c4):
  fin4 = x4.shape[1]
  cn = lambda i: (0, 0)
  return pl.pallas_call(
      _final_body,
      grid=(N_PAD // (4 * NB4),),
      in_specs=[
          pl.BlockSpec((NC, NB4, 128), lambda i: (0, i, 0)),
          pl.BlockSpec((NB4, fin4), lambda i: (i, 0)),
          pl.BlockSpec(w4root.shape, cn),
          pl.BlockSpec((1, 128), cn),
          pl.BlockSpec(w1.shape, cn),
          pl.BlockSpec(c1.shape, cn),
          pl.BlockSpec(w2.shape, cn),
          pl.BlockSpec(c2.shape, cn),
          pl.BlockSpec(w3.shape, cn),
          pl.BlockSpec(c3.shape, cn),
          pl.BlockSpec(w4r.shape, cn),
          pl.BlockSpec(c4.shape, cn),
      ],
      out_specs=pl.BlockSpec((1, 1), cn),
      out_shape=jax.ShapeDtypeStruct((1, 1), jnp.float32),
      scratch_shapes=[pltpu.VMEM((1, 128), jnp.float32)],
  )(parts4, x4, w4root, b4, w1, c1, w2, c2, w3, c3, w4r, c4)


def _prep_u(wk):
  fin = wk.shape[1] // F_OUT
  wk3 = wk.reshape(D_EDGE, fin, F_OUT)
  return jnp.transpose(wk3, (1, 0, 2)).reshape(fin, D_EDGE * F_OUT)


def kernel(x, edge_index, edge_attr, Wk1, bk1, root1, b1, Wk2, bk2, root2, b2,
           Wk3, bk3, root3, b3, W1, bd1, W2, bd2, W3, bd3, W4, bd4):
  src = edge_index[0].astype(jnp.int32)
  dst = edge_index[1].astype(jnp.int32)
  pad = E_PAD - N_EDGES
  src_p = jnp.concatenate([src, jnp.zeros((pad,), jnp.int32)])
  dst_p = jnp.concatenate([dst, jnp.full((pad,), TRASH, jnp.int32)])
  ea4 = _tc_ea_pack(edge_attr)

  src2 = src_p.reshape(E_PAD // CHUNK, CHUNK)
  dst3 = dst_p.reshape(NW, N_CHUNKS, CHUNK)
  zeros_n = jnp.zeros((N_PAD, F_OUT), jnp.float32)

  eye4 = jnp.eye(4, dtype=jnp.float32)
  r_mat = jnp.kron(jnp.eye(D_EDGE, dtype=jnp.float32),
                   jnp.ones((1, F_OUT), jnp.float32))
  s_mat = jnp.kron(jnp.ones((D_EDGE, 1), jnp.float32),
                   jnp.eye(F_OUT, dtype=jnp.float32))
  r4 = jnp.kron(eye4, r_mat).astype(jnp.bfloat16)
  s4 = jnp.kron(eye4, s_mat)

  u41, u42, u43 = _prep_u(Wk1), _prep_u(Wk2), _prep_u(Wk3)
  u41, u42, u43 = (u41.astype(jnp.bfloat16), u42.astype(jnp.bfloat16),
                   u43.astype(jnp.bfloat16))
  bk41 = jnp.kron(eye4, bk1.reshape(D_FEAT, F_OUT))
  bk42 = jnp.kron(eye4, bk2.reshape(F_OUT, F_OUT))
  bk43 = jnp.kron(eye4, bk3.reshape(F_OUT, F_OUT))
  root41 = jnp.kron(eye4, root1)
  root42 = jnp.kron(eye4, root2)
  root43 = jnp.kron(eye4, root3)
  b41 = jnp.tile(b1.reshape(1, F_OUT), (1, 4))
  b42 = jnp.tile(b2.reshape(1, F_OUT), (1, 4))
  b43 = jnp.tile(b3.reshape(1, F_OUT), (1, 4))

  gather128 = _make_sc_gather(D_FEAT)
  gather32 = _make_sc_gather(F_OUT)
  scatter = _make_sc_scatter()

  def layer(h_table, xs_pack, u4, bk4):
    xs = gather128(h_table, src2) if xs_pack is None else (
        gather32(h_table, src2).reshape(E_PAD // 4, 128))
    msg4 = _tc_messages(xs, ea4, u4, bk4, r4, s4)
    msg = msg4.reshape(E_PAD, F_OUT)
    parts = scatter(msg, dst3, zeros_n)
    return parts.reshape(NC, N_PAD // 4, 128)

  x4 = jnp.concatenate([
      x.reshape(N_NODES // 4, 4 * D_FEAT),
      jnp.zeros((N_PAD // 4 - N_NODES // 4, 4 * D_FEAT), jnp.float32)])

  parts1 = layer(x, None, u41, bk41)
  h1_4 = _tc_root(parts1, x4, root41, b41)
  h1 = h1_4.reshape(N_PAD, F_OUT)
  parts2 = layer(h1, True, u42, bk42)
  h2_4 = _tc_root(parts2, h1_4, root42, b42)
  h2 = h2_4.reshape(N_PAD, F_OUT)
  parts3 = layer(h2, True, u43, bk43)

  out = _tc_final(parts3, h2_4, root43, b43,
                  W1, bd1.reshape(1, -1), W2, bd2.reshape(1, -1),
                  W3, bd3.reshape(1, -1), W4.reshape(1, -1),
                  bd4.reshape(1, 1))
  return out


# speedup vs baseline: 1.0634x; 1.0634x over previous
import functools

import jax
import jax.numpy as jnp
from jax import lax
from jax.experimental import pallas as pl
from jax.experimental.pallas import tpu as pltpu
from jax.experimental.pallas import tpu_sc as plsc

N_NODES = 10000
N_EDGES = 160000
D_FEAT = 128
D_EDGE = 16
F_OUT = 32

NC = 2
NS = 16
NW = NC * NS
CHUNK = 128
E_PAD = 163840
N_CHUNKS = E_PAD // (NW * CHUNK)
N_PAD = 10240
TRASH = N_NODES
ROWS_PER_TILE = N_PAD // NS

EB = 8192
NB4 = 512
NBUF = 4


CB_FAST = 60
CB_SLOW = 20


def _make_sc_gather(fin):
  mesh = plsc.VectorSubcoreMesh(core_axis_name="c", subcore_axis_name="s")

  @functools.partial(
      pl.kernel, mesh=mesh,
      out_type=jax.ShapeDtypeStruct((E_PAD, fin), jnp.float32),
      compiler_params=pltpu.CompilerParams(use_tc_tiling_on_sc=False),
      scratch_types=[
          pltpu.VMEM((CB_FAST, CHUNK), jnp.int32),
      ] + [pltpu.VMEM((CHUNK, fin), jnp.float32) for _ in range(NBUF)]
        + [pltpu.SemaphoreType.DMA for _ in range(NBUF)],
  )
  def gather_k(table_hbm, idx_hbm, out_hbm, idx_v, *rest):
    bufs, sems = rest[:NBUF], rest[NBUF:]
    cid = lax.axis_index("c")
    sid = lax.axis_index("s")

    def run(base, count):
      pltpu.sync_copy(idx_hbm.at[pl.ds(base, count)],
                      idx_v.at[pl.ds(0, count)])
      for b in range(NBUF):
        pltpu.async_copy(table_hbm.at[idx_v.at[b]], bufs[b], sems[b])

      def body(g, carry):
        for b in range(NBUF):
          j = g * NBUF + b
          pltpu.make_async_copy(table_hbm.at[idx_v.at[j]], bufs[b],
                                sems[b]).wait()
          pltpu.sync_copy(bufs[b],
                          out_hbm.at[pl.ds((base + j) * CHUNK, CHUNK)])

          @pl.when(j + NBUF < count)
          def _():
            pltpu.async_copy(table_hbm.at[idx_v.at[j + NBUF]], bufs[b],
                             sems[b])

        return carry

      lax.fori_loop(0, count // NBUF, body, 0)

    @pl.when(cid == 1)
    def _():
      run(sid * CB_FAST, CB_FAST)

    @pl.when(cid == 0)
    def _():
      run(NS * CB_FAST + sid * CB_SLOW, CB_SLOW)

  return gather_k


def _make_sc_scatter():
  mesh = plsc.VectorSubcoreMesh(core_axis_name="c", subcore_axis_name="s")

  @functools.partial(
      pl.kernel, mesh=mesh,
      out_type=jax.ShapeDtypeStruct((NC, N_PAD, F_OUT), jnp.float32),
      compiler_params=pltpu.CompilerParams(use_tc_tiling_on_sc=False),
      scratch_types=[
          pltpu.VMEM((N_CHUNKS, CHUNK), jnp.int32),
          pltpu.VMEM_SHARED((N_PAD, F_OUT), jnp.float32),
      ] + [pltpu.VMEM((CHUNK, F_OUT), jnp.float32) for _ in range(NBUF)]
        + [pltpu.SemaphoreType.DMA for _ in range(NBUF)],
  )
  def scatter_k(msg_hbm, dst_hbm, zeros_hbm, out_hbm, idx_v, acc_sh, *rest):
    bufs, sems = rest[:NBUF], rest[NBUF:]
    cid = lax.axis_index("c")
    sid = lax.axis_index("s")
    wid = sid * NC + cid
    r0 = sid * ROWS_PER_TILE
    base = wid * N_CHUNKS

    pltpu.sync_copy(zeros_hbm.at[pl.ds(r0, ROWS_PER_TILE)],
                    acc_sh.at[pl.ds(r0, ROWS_PER_TILE)])
    pltpu.sync_copy(dst_hbm.at[wid], idx_v)
    for b in range(NBUF):
      pltpu.async_copy(msg_hbm.at[pl.ds((base + b) * CHUNK, CHUNK)],
                       bufs[b], sems[b])
    plsc.subcore_barrier()

    def body(g, carry):
      for b in range(NBUF):
        j = g * NBUF + b
        pltpu.make_async_copy(
            msg_hbm.at[pl.ds((base + j) * CHUNK, CHUNK)], bufs[b],
            sems[b]).wait()
        pltpu.sync_copy(bufs[b], acc_sh.at[idx_v.at[j]], add=True)

        @pl.when(j + NBUF < N_CHUNKS)
        def _():
          pltpu.async_copy(
              msg_hbm.at[pl.ds((base + j + NBUF) * CHUNK, CHUNK)],
              bufs[b], sems[b])

      return carry

    lax.fori_loop(0, N_CHUNKS // NBUF, body, 0)

    plsc.subcore_barrier()
    pltpu.sync_copy(acc_sh.at[pl.ds(r0, ROWS_PER_TILE)],
                    out_hbm.at[cid, pl.ds(r0, ROWS_PER_TILE)])

  return scatter_k


EAB = 8192


def _ea_pack_body(ea_ref, out_ref):
  e3 = ea_ref[...].reshape(EAB // 4, 4, D_EDGE)
  out_ref[...] = jnp.concatenate([e3[:, j, :] for j in range(4)], axis=1)


def _tc_ea_pack(ea):
  grid = (N_EDGES + EAB - 1) // EAB
  return pl.pallas_call(
      _ea_pack_body,
      grid=(grid,),
      in_specs=[pl.BlockSpec((EAB, D_EDGE), lambda i: (i, 0))],
      out_specs=pl.BlockSpec((EAB // 4, 4 * D_EDGE), lambda i: (i, 0)),
      out_shape=jax.ShapeDtypeStruct((E_PAD // 4, 4 * D_EDGE), jnp.float32),
  )(ea)


def _msg_body(fin, xs_ref, ea_ref, u_ref, bk_ref, r_ref, s_ref, out_ref):
  xs4 = xs_ref[...]
  if fin == 128:
    xs4 = xs4.reshape(EB // 4, 4 * fin)
  eab = ea_ref[...]
  acc = jnp.dot(xs4, bk_ref[...], preferred_element_type=jnp.float32)
  for g in range(4):
    y = jnp.dot(xs4[:, g * fin:(g + 1) * fin], u_ref[...],
                preferred_element_type=jnp.float32)
    a = jnp.dot(eab, r_ref[:, g * 512:(g + 1) * 512],
                preferred_element_type=jnp.float32)
    acc = acc + jnp.dot(a * y, s_ref[pl.ds(g * 512, 512), :],
                        preferred_element_type=jnp.float32)
  out_ref[...] = acc


def _tc_messages(xs, ea4, u, bk4, r4, s4):
  fin = u.shape[0]
  xs_block = (EB, 128) if fin == 128 else (EB // 4, 128)
  cn = lambda i: (0, 0)
  return pl.pallas_call(
      functools.partial(_msg_body, fin),
      grid=(E_PAD // EB,),
      in_specs=[
          pl.BlockSpec(xs_block, lambda i: (i, 0)),
          pl.BlockSpec((EB // 4, 64), lambda i: (i, 0)),
          pl.BlockSpec(u.shape, cn),
          pl.BlockSpec(bk4.shape, cn),
          pl.BlockSpec(r4.shape, cn),
          pl.BlockSpec(s4.shape, cn),
      ],
      out_specs=pl.BlockSpec((EB // 4, 128), lambda i: (i, 0)),
      out_shape=jax.ShapeDtypeStruct((E_PAD // 4, 128), jnp.float32),
      compiler_params=pltpu.CompilerParams(vmem_limit_bytes=100 * 1024 * 1024),
  )(xs, ea4, u, bk4, r4, s4)


def _root_body(p_ref, x_ref, w_ref, b_ref, out_ref):
  agg = p_ref[0] + p_ref[1]
  z = agg + jnp.dot(x_ref[...], w_ref[...],
                    preferred_element_type=jnp.float32) + b_ref[...]
  out_ref[...] = jnp.maximum(z, 0.0)


def _tc_root(parts4, x4, w4root, b4):
  fin4 = x4.shape[1]
  return pl.pallas_call(
      _root_body,
      grid=(N_PAD // (4 * NB4),),
      in_specs=[
          pl.BlockSpec((NC, NB4, 128), lambda i: (0, i, 0)),
          pl.BlockSpec((NB4, fin4), lambda i: (i, 0)),
          pl.BlockSpec(w4root.shape, lambda i: (0, 0)),
          pl.BlockSpec((1, 128), lambda i: (0, 0)),
      ],
      out_specs=pl.BlockSpec((NB4, 128), lambda i: (i, 0)),
      out_shape=jax.ShapeDtypeStruct((N_PAD // 4, 128), jnp.float32),
  )(parts4, x4, w4root, b4)


def _final_body(p_ref, x_ref, w_ref, b_ref, w1_ref, c1_ref, w2_ref, c2_ref,
                w3_ref, c3_ref, w4_ref, c4_ref, out_ref, acc_ref):
  i = pl.program_id(0)
  agg = p_ref[0] + p_ref[1]
  h = jnp.maximum(
      agg + jnp.dot(x_ref[...], w_ref[...],
                    preferred_element_type=jnp.float32) + b_ref[...], 0.0)
  row = i * NB4 + lax.broadcasted_iota(jnp.int32, (NB4, 1), 0)
  h = jnp.where(row < N_NODES // 4, h, 0.0)
  psum = jnp.sum(h, axis=0, keepdims=True)

  @pl.when(i == 0)
  def _():
    acc_ref[...] = jnp.zeros_like(acc_ref)

  acc_ref[...] += psum

  @pl.when(i == pl.num_programs(0) - 1)
  def _():
    a = acc_ref[...]
    pooled = (a[:, 0:32] + a[:, 32:64] + a[:, 64:96] + a[:, 96:128])
    z = jnp.maximum(jnp.dot(pooled, w1_ref[...],
                            preferred_element_type=jnp.float32) + c1_ref[...], 0.0)
    z = jnp.maximum(jnp.dot(z, w2_ref[...],
                            preferred_element_type=jnp.float32) + c2_ref[...], 0.0)
    z = jnp.maximum(jnp.dot(z, w3_ref[...],
                            preferred_element_type=jnp.float32) + c3_ref[...], 0.0)
    out_ref[...] = jnp.sum(z * w4_ref[...]).reshape(1, 1) + c4_ref[...]


def _tc_final(parts4, x4, w4root, b4, w1, c1, w2, c2, w3, c3, w4r, c4):
  fin4 = x4.shape[1]
  cn = lambda i: (0, 0)
  return pl.pallas_call(
      _final_body,
      grid=(N_PAD // (4 * NB4),),
      in_specs=[
          pl.BlockSpec((NC, NB4, 128), lambda i: (0, i, 0)),
          pl.BlockSpec((NB4, fin4), lambda i: (i, 0)),
          pl.BlockSpec(w4root.shape, cn),
          pl.BlockSpec((1, 128), cn),
          pl.BlockSpec(w1.shape, cn),
          pl.BlockSpec(c1.shape, cn),
          pl.BlockSpec(w2.shape, cn),
          pl.BlockSpec(c2.shape, cn),
          pl.BlockSpec(w3.shape, cn),
          pl.BlockSpec(c3.shape, cn),
          pl.BlockSpec(w4r.shape, cn),
          pl.BlockSpec(c4.shape, cn),
      ],
      out_specs=pl.BlockSpec((1, 1), cn),
      out_shape=jax.ShapeDtypeStruct((1, 1), jnp.float32),
      scratch_shapes=[pltpu.VMEM((1, 128), jnp.float32)],
  )(parts4, x4, w4root, b4, w1, c1, w2, c2, w3, c3, w4r, c4)


def _prep_u(wk):
  fin = wk.shape[1] // F_OUT
  wk3 = wk.reshape(D_EDGE, fin, F_OUT)
  return jnp.transpose(wk3, (1, 0, 2)).reshape(fin, D_EDGE * F_OUT)


def kernel(x, edge_index, edge_attr, Wk1, bk1, root1, b1, Wk2, bk2, root2, b2,
           Wk3, bk3, root3, b3, W1, bd1, W2, bd2, W3, bd3, W4, bd4):
  src = edge_index[0].astype(jnp.int32)
  dst = edge_index[1].astype(jnp.int32)
  pad = E_PAD - N_EDGES
  src_p = jnp.concatenate([src, jnp.zeros((pad,), jnp.int32)])
  dst_p = jnp.concatenate([dst, jnp.full((pad,), TRASH, jnp.int32)])
  ea4 = _tc_ea_pack(edge_attr)

  src2 = src_p.reshape(E_PAD // CHUNK, CHUNK)
  dst3 = dst_p.reshape(NW, N_CHUNKS, CHUNK)
  zeros_n = jnp.zeros((N_PAD, F_OUT), jnp.float32)

  eye4 = jnp.eye(4, dtype=jnp.float32)
  r_mat = jnp.kron(jnp.eye(D_EDGE, dtype=jnp.float32),
                   jnp.ones((1, F_OUT), jnp.float32))
  s_mat = jnp.kron(jnp.ones((D_EDGE, 1), jnp.float32),
                   jnp.eye(F_OUT, dtype=jnp.float32))
  r4 = jnp.kron(eye4, r_mat)
  s4 = jnp.kron(eye4, s_mat)

  u41, u42, u43 = _prep_u(Wk1), _prep_u(Wk2), _prep_u(Wk3)
  bk41 = jnp.kron(eye4, bk1.reshape(D_FEAT, F_OUT))
  bk42 = jnp.kron(eye4, bk2.reshape(F_OUT, F_OUT))
  bk43 = jnp.kron(eye4, bk3.reshape(F_OUT, F_OUT))
  root41 = jnp.kron(eye4, root1)
  root42 = jnp.kron(eye4, root2)
  root43 = jnp.kron(eye4, root3)
  b41 = jnp.tile(b1.reshape(1, F_OUT), (1, 4))
  b42 = jnp.tile(b2.reshape(1, F_OUT), (1, 4))
  b43 = jnp.tile(b3.reshape(1, F_OUT), (1, 4))

  gather128 = _make_sc_gather(D_FEAT)
  gather32 = _make_sc_gather(F_OUT)
  scatter = _make_sc_scatter()

  def layer(h_table, xs_pack, u4, bk4):
    xs = gather128(h_table, src2) if xs_pack is None else (
        gather32(h_table, src2).reshape(E_PAD // 4, 128))
    msg4 = _tc_messages(xs, ea4, u4, bk4, r4, s4)
    msg = msg4.reshape(E_PAD, F_OUT)
    parts = scatter(msg, dst3, zeros_n)
    return parts.reshape(NC, N_PAD // 4, 128)

  x4 = jnp.concatenate([
      x.reshape(N_NODES // 4, 4 * D_FEAT),
      jnp.zeros((N_PAD // 4 - N_NODES // 4, 4 * D_FEAT), jnp.float32)])

  parts1 = layer(x, None, u41, bk41)
  h1_4 = _tc_root(parts1, x4, root41, b41)
  h1 = h1_4.reshape(N_PAD, F_OUT)
  parts2 = layer(h1, True, u42, bk42)
  h2_4 = _tc_root(parts2, h1_4, root42, b42)
  h2 = h2_4.reshape(N_PAD, F_OUT)
  parts3 = layer(h2, True, u43, bk43)

  out = _tc_final(parts3, h2_4, root43, b43,
                  W1, bd1.reshape(1, -1), W2, bd2.reshape(1, -1),
                  W3, bd3.reshape(1, -1), W4.reshape(1, -1),
                  bd4.reshape(1, 1))
  return out
